# Initial kernel scaffold; baseline (speedup 1.0000x reference)
import jax
import jax.numpy as jnp
from jax.experimental import pallas as pl


def kernel(x, edge_index, W, b):
    raise NotImplementedError("write your pallas kernel here")


# speedup vs baseline: 10.0252x; 10.0252x over previous
import functools

import jax
import jax.numpy as jnp
from jax import lax
from jax.experimental import pallas as pl
from jax.experimental.pallas import tpu as pltpu
from jax.experimental.pallas import tpu_sc as plsc

NC = 2
NS = 16
L = 16
G = 128

F32 = jnp.float32
I32 = jnp.int32


def _round_up(a, m):
  return (a + m - 1) // m * m


def _make_deg(E, NPAD):
  EC = E // (NC * NS)
  ZC = NPAD // NS
  mesh = plsc.VectorSubcoreMesh(core_axis_name="c", subcore_axis_name="s")

  @functools.partial(
      pl.kernel,
      mesh=mesh,
      out_type=jax.ShapeDtypeStruct((NC * NPAD,), F32),
      scratch_types=[
          pltpu.VMEM((EC,), F32),
          pltpu.VMEM((EC,), I32),
          pltpu.VMEM((ZC,), F32),
          pltpu.VMEM_SHARED((NPAD,), F32),
      ],
  )
  def deg_kernel(col_hbm, deg_hbm, onesv, idxv, zv, deg_sh):
    c = lax.axis_index("c")
    s = lax.axis_index("s")
    wid = c * NS + s

    one16 = jnp.full((L,), 1.0, F32)
    zero16 = jnp.zeros((L,), F32)

    def fill_ones(i, _):
      onesv[pl.ds(i * L, L)] = one16
      return 0
    lax.fori_loop(0, EC // L, fill_ones, 0)
    if EC % L:
      onesv[pl.ds(EC - L, L)] = one16

    def fill_zero(i, _):
      zv[pl.ds(i * L, L)] = zero16
      return 0
    lax.fori_loop(0, ZC // L, fill_zero, 0)

    pltpu.sync_copy(zv, deg_sh.at[pl.ds(s * ZC, ZC)])
    plsc.subcore_barrier()

    pltpu.sync_copy(col_hbm.at[pl.ds(wid * EC, EC)], idxv)
    pltpu.sync_copy(onesv, deg_sh.at[idxv], add=True)
    plsc.subcore_barrier()

    pltpu.sync_copy(deg_sh.at[pl.ds(s * ZC, ZC)], zv)
    pltpu.sync_copy(zv, deg_hbm.at[pl.ds(c * NPAD + s * ZC, ZC)])

  return deg_kernel


def _make_linear(N, CIN, COUT, NPAD, BN):
  H = COUT // 4
  grid = ((N + BN - 1) // BN,)

  def lin_kernel(x_ref, wt_ref, b_ref, dg_ref, tab_ref):
    h = jnp.dot(x_ref[...], wt_ref[...], preferred_element_type=F32)
    h = h + b_ref[...]
    dis = lax.rsqrt(1.0 + dg_ref[0] + dg_ref[1])
    h = h * dis[:, None]
    for q in range(4):
      tab_ref[q] = h[:, q * H:(q + 1) * H]

  return pl.pallas_call(
      lin_kernel,
      grid=grid,
      in_specs=[
          pl.BlockSpec((BN, CIN), lambda i: (i, 0)),
          pl.BlockSpec((CIN, COUT), lambda i: (0, 0)),
          pl.BlockSpec((1, COUT), lambda i: (0, 0)),
          pl.BlockSpec((NC, BN), lambda i: (0, i)),
      ],
      out_specs=pl.BlockSpec((4, BN, H), lambda i: (0, i, 0)),
      out_shape=jax.ShapeDtypeStruct((4, N, H), F32),
  )


def _make_scatter(N, H, NPAD, NBT):
  NB = NBT // NS
  ZR = NPAD // NS
  ZB = ZR // G
  mesh = plsc.VectorSubcoreMesh(core_axis_name="c", subcore_axis_name="s")

  @functools.partial(
      pl.kernel,
      mesh=mesh,
      out_type=jax.ShapeDtypeStruct((4, NPAD, H), F32),
      compiler_params=pltpu.CompilerParams(use_tc_tiling_on_sc=False),
      scratch_types=[
          pltpu.VMEM((NB, G), I32),
          pltpu.VMEM((NB, G), I32),
          pltpu.VMEM((G, H), F32),
          pltpu.VMEM((G, H), F32),
          pltpu.VMEM_SHARED((NPAD, H), F32),
          pltpu.SemaphoreType.DMA,
          pltpu.SemaphoreType.DMA,
      ],
  )
  def scat_kernel(tab_hbm, row_hbm, colb_hbm, acc_hbm,
                  rowv, colv, gbuf0, gbuf1, acc_sh, sem0, sem1):
    c = lax.axis_index("c")
    s = lax.axis_index("s")

    zero16 = jnp.zeros((L,), F32)

    pltpu.sync_copy(row_hbm.at[pl.ds(s * NB, NB), :], rowv)

    for p in range(2):
      q = c * 2 + p

      def zrow(i, _):
        for k in range(H // L):
          gbuf0[i, pl.ds(k * L, L)] = zero16
        return 0
      lax.fori_loop(0, G, zrow, 0)
      for m in range(ZB):
        pltpu.sync_copy(gbuf0, acc_sh.at[pl.ds(s * ZR + m * G, G), :])

      pltpu.sync_copy(colb_hbm.at[q, pl.ds(s * NB, NB), :], colv)
      plsc.subcore_barrier()

      pltpu.async_copy(tab_hbm.at[colv.at[0]], gbuf0, sem0)

      def body(j, _):
        even = lax.rem(j, 2) == 0

        @pl.when(even)
        def _():
          @pl.when(j + 1 < NB)
          def _():
            pltpu.async_copy(tab_hbm.at[colv.at[j + 1]], gbuf1, sem1)
          pltpu.make_async_copy(tab_hbm.at[colv.at[j]], gbuf0, sem0).wait()
          pltpu.sync_copy(gbuf0, acc_sh.at[rowv.at[j]], add=True)

        @pl.when(jnp.logical_not(even))
        def _():
          @pl.when(j + 1 < NB)
          def _():
            pltpu.async_copy(tab_hbm.at[colv.at[j + 1]], gbuf0, sem0)
          pltpu.make_async_copy(tab_hbm.at[colv.at[j]], gbuf1, sem1).wait()
          pltpu.sync_copy(gbuf1, acc_sh.at[rowv.at[j]], add=True)

        return 0

      lax.fori_loop(0, NB, body, 0)
      plsc.subcore_barrier()

      for m in range(ZB):
        pltpu.sync_copy(acc_sh.at[pl.ds(s * ZR + m * G, G), :], gbuf0)
        pltpu.sync_copy(gbuf0, acc_hbm.at[q, pl.ds(s * ZR + m * G, G), :])

  return scat_kernel


def _make_final(N, COUT, NPAD, BN):
  H = COUT // 4
  grid = ((N + BN - 1) // BN,)

  def fin_kernel(acc_ref, tab_ref, dg_ref, out_ref):
    dis = lax.rsqrt(1.0 + dg_ref[0] + dg_ref[1])[:, None]
    for q in range(4):
      out_ref[:, q * H:(q + 1) * H] = dis * (acc_ref[q] + tab_ref[q])

  return pl.pallas_call(
      fin_kernel,
      grid=grid,
      in_specs=[
          pl.BlockSpec((4, BN, H), lambda i: (0, i, 0)),
          pl.BlockSpec((4, BN, H), lambda i: (0, i, 0)),
          pl.BlockSpec((NC, BN), lambda i: (0, i)),
      ],
      out_specs=pl.BlockSpec((BN, COUT), lambda i: (i, 0)),
      out_shape=jax.ShapeDtypeStruct((N, COUT), F32),
  )


@jax.jit
def kernel(x, edge_index, W, b):
  N, CIN = x.shape
  COUT = W.shape[0]
  H = COUT // 4
  E = edge_index.shape[1]

  NPAD = _round_up(N + 1, NS * G)
  E2 = _round_up(E, NS * G * 8)
  NBT = E2 // G
  BN = 512

  row = edge_index[0]
  col = edge_index[1]
  pad = E2 - E
  rowp = jnp.concatenate([row, jnp.full((pad,), N, I32)]).reshape(NBT, G)
  colp = jnp.concatenate([col, jnp.zeros((pad,), I32)]).reshape(NBT, G)
  colb = jnp.stack([colp + q * N for q in range(4)])

  degsum = _make_deg(E, NPAD)(col).reshape(NC, NPAD)
  table = _make_linear(N, CIN, COUT, NPAD, BN)(
      x, W.T, b[None, :], degsum)
  acc = _make_scatter(N, H, NPAD, NBT)(
      table.reshape(4 * N, H), rowp, colb)
  out = _make_final(N, COUT, NPAD, BN)(acc, table, degsum)
  return out


# speedup vs baseline: 10.5054x; 1.0479x over previous
import functools

import jax
import jax.numpy as jnp
from jax import lax
from jax.experimental import pallas as pl
from jax.experimental.pallas import tpu as pltpu
from jax.experimental.pallas import tpu_sc as plsc

NC = 2
NS = 16
L = 16
G = 128

F32 = jnp.float32
I32 = jnp.int32


def _round_up(a, m):
  return (a + m - 1) // m * m


def _make_deg(E, NPAD):
  EC = E // (NC * NS)
  ZC = NPAD // NS
  mesh = plsc.VectorSubcoreMesh(core_axis_name="c", subcore_axis_name="s")

  @functools.partial(
      pl.kernel,
      mesh=mesh,
      out_type=jax.ShapeDtypeStruct((NC * NPAD,), F32),
      scratch_types=[
          pltpu.VMEM((EC,), F32),
          pltpu.VMEM((EC,), I32),
          pltpu.VMEM((ZC,), F32),
          pltpu.VMEM_SHARED((NPAD,), F32),
      ],
  )
  def deg_kernel(col_hbm, deg_hbm, onesv, idxv, zv, deg_sh):
    c = lax.axis_index("c")
    s = lax.axis_index("s")
    wid = c * NS + s

    one16 = jnp.full((L,), 1.0, F32)
    zero16 = jnp.zeros((L,), F32)

    def fill_ones(i, _):
      onesv[pl.ds(i * L, L)] = one16
      return 0
    lax.fori_loop(0, EC // L, fill_ones, 0)
    if EC % L:
      onesv[pl.ds(EC - L, L)] = one16

    def fill_zero(i, _):
      zv[pl.ds(i * L, L)] = zero16
      return 0
    lax.fori_loop(0, ZC // L, fill_zero, 0)

    pltpu.sync_copy(zv, deg_sh.at[pl.ds(s * ZC, ZC)])
    plsc.subcore_barrier()

    pltpu.sync_copy(col_hbm.at[pl.ds(wid * EC, EC)], idxv)
    pltpu.sync_copy(onesv, deg_sh.at[idxv], add=True)
    plsc.subcore_barrier()

    pltpu.sync_copy(deg_sh.at[pl.ds(s * ZC, ZC)], zv)
    pltpu.sync_copy(zv, deg_hbm.at[pl.ds(c * NPAD + s * ZC, ZC)])

  return deg_kernel


def _make_linear(N, CIN, COUT, NPAD, BN):
  H = COUT // 4
  grid = ((N + BN - 1) // BN,)

  def lin_kernel(x_ref, wt_ref, b_ref, dg_ref, tab_ref):
    h = jnp.dot(x_ref[...], wt_ref[...], preferred_element_type=F32)
    h = h + b_ref[...]
    dis = lax.rsqrt(1.0 + dg_ref[0] + dg_ref[1])
    h = h * dis[:, None]
    for q in range(4):
      tab_ref[q] = h[:, q * H:(q + 1) * H]

  return pl.pallas_call(
      lin_kernel,
      grid=grid,
      in_specs=[
          pl.BlockSpec((BN, CIN), lambda i: (i, 0)),
          pl.BlockSpec((CIN, COUT), lambda i: (0, 0)),
          pl.BlockSpec((1, COUT), lambda i: (0, 0)),
          pl.BlockSpec((NC, BN), lambda i: (0, i)),
      ],
      out_specs=pl.BlockSpec((4, BN, H), lambda i: (0, i, 0)),
      out_shape=jax.ShapeDtypeStruct((4, N, H), F32),
  )


def _make_scatter(N, H, NPAD, NBT):
  NB = NBT // NS
  ZR = NPAD // NS
  ZB = ZR // G
  mesh = plsc.VectorSubcoreMesh(core_axis_name="c", subcore_axis_name="s")

  @functools.partial(
      pl.kernel,
      mesh=mesh,
      out_type=jax.ShapeDtypeStruct((4, NPAD, H), F32),
      compiler_params=pltpu.CompilerParams(use_tc_tiling_on_sc=False),
      scratch_types=[
          pltpu.VMEM((NB, G), I32),
          pltpu.VMEM((NB, G), I32),
          [pltpu.VMEM((G, H), F32)] * 4,
          pltpu.VMEM_SHARED((NPAD, H), F32),
          [pltpu.SemaphoreType.DMA] * 4,
          [pltpu.SemaphoreType.DMA] * 4,
      ],
  )
  def scat_kernel(tab_hbm, row_hbm, colb_hbm, acc_hbm,
                  rowv, colv, bufs, acc_sh, gsems, ssems):
    c = lax.axis_index("c")
    s = lax.axis_index("s")

    zero16 = jnp.zeros((L,), F32)

    pltpu.sync_copy(row_hbm.at[pl.ds(s * NB, NB), :], rowv)

    NBUF = 4
    D = 2

    def gather(j, k):
      pltpu.async_copy(tab_hbm.at[colv.at[j]], bufs[k], gsems[k])

    def gather_wait(j, k):
      pltpu.make_async_copy(tab_hbm.at[colv.at[j]], bufs[k], gsems[k]).wait()

    def scat(j, k):
      pltpu.async_copy(bufs[k], acc_sh.at[rowv.at[j]], ssems[k], add=True)

    def scat_wait(j, k):
      pltpu.make_async_copy(bufs[k], acc_sh.at[rowv.at[j]], ssems[k]).wait()

    for p in range(2):
      q = c * 2 + p

      def zrow(i, _):
        for k in range(H // L):
          bufs[0][i, pl.ds(k * L, L)] = zero16
        return 0
      lax.fori_loop(0, G, zrow, 0)
      for m in range(ZB):
        pltpu.sync_copy(bufs[0], acc_sh.at[pl.ds(s * ZR + m * G, G), :])

      pltpu.sync_copy(colb_hbm.at[q, pl.ds(s * NB, NB), :], colv)
      plsc.subcore_barrier()

      for j in range(D):
        gather(j, j % NBUF)

      def body(r, _):
        for k in range(NBUF):
          j = r * NBUF + k
          jg = j + D
          js = jg - NBUF
          kg = (k + D) % NBUF

          @pl.when(jnp.logical_and(js >= 0, jg < NB))
          def _():
            scat_wait(js, kg)
          @pl.when(jg < NB)
          def _():
            gather(jg, kg)

          gather_wait(j, k)
          scat(j, k)
        return 0

      lax.fori_loop(0, NB // NBUF, body, 0)
      for j in range(NB - NBUF, NB):
        scat_wait(j, j % NBUF)
      plsc.subcore_barrier()

      for m in range(ZB):
        pltpu.sync_copy(acc_sh.at[pl.ds(s * ZR + m * G, G), :], bufs[0])
        pltpu.sync_copy(bufs[0], acc_hbm.at[q, pl.ds(s * ZR + m * G, G), :])

  return scat_kernel


def _make_final(N, COUT, NPAD, BN):
  H = COUT // 4
  grid = ((N + BN - 1) // BN,)

  def fin_kernel(acc_ref, tab_ref, dg_ref, out_ref):
    dis = lax.rsqrt(1.0 + dg_ref[0] + dg_ref[1])[:, None]
    for q in range(4):
      out_ref[:, q * H:(q + 1) * H] = dis * (acc_ref[q] + tab_ref[q])

  return pl.pallas_call(
      fin_kernel,
      grid=grid,
      in_specs=[
          pl.BlockSpec((4, BN, H), lambda i: (0, i, 0)),
          pl.BlockSpec((4, BN, H), lambda i: (0, i, 0)),
          pl.BlockSpec((NC, BN), lambda i: (0, i)),
      ],
      out_specs=pl.BlockSpec((BN, COUT), lambda i: (i, 0)),
      out_shape=jax.ShapeDtypeStruct((N, COUT), F32),
  )


@jax.jit
def kernel(x, edge_index, W, b):
  N, CIN = x.shape
  COUT = W.shape[0]
  H = COUT // 4
  E = edge_index.shape[1]

  NPAD = _round_up(N + 1, NS * G)
  E2 = _round_up(E, NS * G * 8)
  NBT = E2 // G
  BN = 512

  row = edge_index[0]
  col = edge_index[1]
  pad = E2 - E
  rowp = jnp.concatenate([row, jnp.full((pad,), N, I32)]).reshape(NBT, G)
  colp = jnp.concatenate([col, jnp.zeros((pad,), I32)]).reshape(NBT, G)
  colb = jnp.stack([colp + q * N for q in range(4)])

  degsum = _make_deg(E, NPAD)(col).reshape(NC, NPAD)
  table = _make_linear(N, CIN, COUT, NPAD, BN)(
      x, W.T, b[None, :], degsum)
  acc = _make_scatter(N, H, NPAD, NBT)(
      table.reshape(4 * N, H), rowp, colb)
  out = _make_final(N, COUT, NPAD, BN)(acc, table, degsum)
  return out
